# Initial kernel scaffold; baseline (speedup 1.0000x reference)
import functools

import jax
import jax.numpy as jnp
from jax.experimental import pallas as pl

_N = 10000
_E = 320000
_HID = 128
_NCONV = 3


def _msg_body(s_ref, m_ref, o_ref):
    s = s_ref[...]
    m = m_ref[...]
    o_ref[...] = jax.nn.sigmoid(s) * jax.nn.softplus(m)


def _msg(s, m, block=8192):
    e = s.shape[0]
    grid = (e // block,)
    return pl.pallas_call(
        _msg_body,
        grid=grid,
        in_specs=[
            pl.BlockSpec((block, _HID), lambda i: (i, 0)),
            pl.BlockSpec((block, _HID), lambda i: (i, 0)),
        ],
        out_specs=pl.BlockSpec((block, _HID), lambda i: (i, 0)),
        out_shape=jax.ShapeDtypeStruct((e, _HID), jnp.float32),
    )(s, m)


def kernel(x, edge_index, edge_attr, W_emb, b_emb, Ws, bs, Wm, bm, gamma,
           beta, W_fc, b_fc, W_out, b_out):
    src = edge_index[0]
    dst = edge_index[1]
    h = x @ W_emb.T + b_emb
    for i in range(_NCONV):
        Ws_src = Ws[i][:, :_HID]
        Ws_dst = Ws[i][:, _HID:2 * _HID]
        Ws_edge = Ws[i][:, 2 * _HID:]
        Wm_src = Wm[i][:, :_HID]
        Wm_dst = Wm[i][:, _HID:2 * _HID]
        Wm_edge = Wm[i][:, 2 * _HID:]
        As = h @ Ws_src.T
        Bs = h @ Ws_dst.T
        Am = h @ Wm_src.T
        Bm = h @ Wm_dst.T
        Cs = edge_attr @ Ws_edge.T + bs[i]
        Cm = edge_attr @ Wm_edge.T + bm[i]
        s = As[src] + Bs[dst] + Cs
        m = Am[src] + Bm[dst] + Cm
        msg = _msg(s, m)
        h = h + jax.ops.segment_sum(msg, dst, num_segments=_N)
        mean = jnp.mean(h, axis=0)
        var = jnp.var(h, axis=0)
        h = (h - mean) / jnp.sqrt(var + 1e-5) * gamma[i] + beta[i]
    graph_feat = jnp.sum(h, axis=0, keepdims=True)
    graph_feat = graph_feat @ W_fc.T + b_fc
    out = graph_feat @ W_out.T + b_out
    return out


# speedup vs baseline: 1.0025x; 1.0025x over previous
import jax
import jax.numpy as jnp
from jax.experimental import pallas as pl

_N = 10000
_HID = 128
_NCONV = 3


def _id_body(x_ref, o_ref):
    o_ref[...] = x_ref[...]


def _id(x):
    return pl.pallas_call(
        _id_body,
        out_shape=jax.ShapeDtypeStruct(x.shape, x.dtype),
    )(x)


def kernel(x, edge_index, edge_attr, W_emb, b_emb, Ws, bs, Wm, bm, gamma,
           beta, W_fc, b_fc, W_out, b_out):
    x = _id(x)
    src = edge_index[0]
    dst = edge_index[1]
    h = x @ W_emb.T + b_emb
    for i in range(_NCONV):
        z = jnp.concatenate([h[src], h[dst], edge_attr], axis=1)
        gated_z = jax.nn.sigmoid(z @ Ws[i].T + bs[i])
        message_z = jax.nn.softplus(z @ Wm[i].T + bm[i])
        h = h + jax.ops.segment_sum(gated_z * message_z, dst, num_segments=_N)
        mean = jnp.mean(h, axis=0)
        var = jnp.var(h, axis=0)
        h = (h - mean) / jnp.sqrt(var + 1e-5) * gamma[i] + beta[i]
    graph_feat = jnp.sum(h, axis=0, keepdims=True)
    graph_feat = graph_feat @ W_fc.T + b_fc
    out = graph_feat @ W_out.T + b_out
    return out


# speedup vs baseline: 1.0254x; 1.0228x over previous
import jax
import jax.numpy as jnp
from jax.experimental import pallas as pl

_N = 10000
_E = 320000
_HID = 128
_NCONV = 3
_BE = 4000


def _msg_body(hs_ref, hd_ref, at_ref, ws_ref, wm_ref, bs_ref, bm_ref, o_ref):
    z256 = jnp.concatenate([hs_ref[...], hd_ref[...]], axis=1)
    a16 = at_ref[...]
    ys = jnp.dot(z256, ws_ref[:2 * _HID]) + jnp.dot(a16, ws_ref[2 * _HID:]) + bs_ref[...]
    ym = jnp.dot(z256, wm_ref[:2 * _HID]) + jnp.dot(a16, wm_ref[2 * _HID:]) + bm_ref[...]
    o_ref[...] = jax.nn.sigmoid(ys) * jax.nn.softplus(ym)


def _msg(hs, hd, attr, ws_t, wm_t, bsi, bmi):
    e = hs.shape[0]
    zdim = ws_t.shape[0]
    return pl.pallas_call(
        _msg_body,
        grid=(e // _BE,),
        in_specs=[
            pl.BlockSpec((_BE, _HID), lambda i: (i, 0)),
            pl.BlockSpec((_BE, _HID), lambda i: (i, 0)),
            pl.BlockSpec((_BE, zdim - 2 * _HID), lambda i: (i, 0)),
            pl.BlockSpec((zdim, _HID), lambda i: (0, 0)),
            pl.BlockSpec((zdim, _HID), lambda i: (0, 0)),
            pl.BlockSpec((1, _HID), lambda i: (0, 0)),
            pl.BlockSpec((1, _HID), lambda i: (0, 0)),
        ],
        out_specs=pl.BlockSpec((_BE, _HID), lambda i: (i, 0)),
        out_shape=jax.ShapeDtypeStruct((e, _HID), jnp.float32),
    )(hs, hd, attr, ws_t, wm_t, bsi.reshape(1, _HID), bmi.reshape(1, _HID))


def kernel(x, edge_index, edge_attr, W_emb, b_emb, Ws, bs, Wm, bm, gamma,
           beta, W_fc, b_fc, W_out, b_out):
    src = edge_index[0]
    dst = edge_index[1]
    h = x @ W_emb.T + b_emb
    for i in range(_NCONV):
        hs = h[src]
        hd = h[dst]
        msg = _msg(hs, hd, edge_attr, Ws[i].T, Wm[i].T, bs[i], bm[i])
        h = h + jax.ops.segment_sum(msg, dst, num_segments=_N)
        mean = jnp.mean(h, axis=0)
        var = jnp.var(h, axis=0)
        h = (h - mean) / jnp.sqrt(var + 1e-5) * gamma[i] + beta[i]
    graph_feat = jnp.sum(h, axis=0, keepdims=True)
    graph_feat = graph_feat @ W_fc.T + b_fc
    out = graph_feat @ W_out.T + b_out
    return out


# speedup vs baseline: 1.9302x; 1.8824x over previous
import functools

import jax
import jax.numpy as jnp
from jax import lax
from jax.experimental import pallas as pl
from jax.experimental.pallas import tpu as pltpu
from jax.experimental.pallas import tpu_sc as plsc

_N = 10000
_E = 320000
_HID = 128
_NCONV = 3
_BE = 4000

_NW = 32
_EPW = _E // _NW
_CH = 400
_NCH = _EPW // _CH


def _gather_body(h_hbm, src_hbm, dst_hbm, hs_out, hd_out,
                 sidx, didx, srows, drows, sem_s, sem_d):
    wid = lax.axis_index("s") * 2 + lax.axis_index("c")
    base = wid * _EPW
    pltpu.sync_copy(src_hbm.at[pl.ds(base, _EPW)], sidx)
    pltpu.sync_copy(dst_hbm.at[pl.ds(base, _EPW)], didx)

    def body(j, c):
        off = j * _CH
        cp_s = pltpu.async_copy(h_hbm.at[sidx.at[pl.ds(off, _CH)]], srows, sem_s)
        cp_d = pltpu.async_copy(h_hbm.at[didx.at[pl.ds(off, _CH)]], drows, sem_d)
        cp_s.wait()
        cp_d.wait()
        pltpu.sync_copy(srows, hs_out.at[pl.ds(base + off, _CH)])
        pltpu.sync_copy(drows, hd_out.at[pl.ds(base + off, _CH)])
        return c

    lax.fori_loop(0, _NCH, body, 0)


@functools.partial(
    pl.kernel,
    mesh=plsc.VectorSubcoreMesh(core_axis_name="c", subcore_axis_name="s"),
    out_type=[
        jax.ShapeDtypeStruct((_E, _HID), jnp.float32),
        jax.ShapeDtypeStruct((_E, _HID), jnp.float32),
    ],
    scratch_types=[
        pltpu.VMEM((_EPW,), jnp.int32),
        pltpu.VMEM((_EPW,), jnp.int32),
        pltpu.VMEM((_CH, _HID), jnp.float32),
        pltpu.VMEM((_CH, _HID), jnp.float32),
        pltpu.SemaphoreType.DMA,
        pltpu.SemaphoreType.DMA,
    ],
)
def _gather2(h_hbm, src_hbm, dst_hbm, hs_out, hd_out,
             sidx, didx, srows, drows, sem_s, sem_d):
    _gather_body(h_hbm, src_hbm, dst_hbm, hs_out, hd_out,
                 sidx, didx, srows, drows, sem_s, sem_d)


def _msg_body(hs_ref, hd_ref, at_ref, ws_ref, wm_ref, bs_ref, bm_ref, o_ref):
    z256 = jnp.concatenate([hs_ref[...], hd_ref[...]], axis=1)
    a16 = at_ref[...]
    ys = jnp.dot(z256, ws_ref[:2 * _HID]) + jnp.dot(a16, ws_ref[2 * _HID:]) + bs_ref[...]
    ym = jnp.dot(z256, wm_ref[:2 * _HID]) + jnp.dot(a16, wm_ref[2 * _HID:]) + bm_ref[...]
    o_ref[...] = jax.nn.sigmoid(ys) * jax.nn.softplus(ym)


def _msg(hs, hd, attr, ws_t, wm_t, bsi, bmi):
    e = hs.shape[0]
    zdim = ws_t.shape[0]
    return pl.pallas_call(
        _msg_body,
        grid=(e // _BE,),
        in_specs=[
            pl.BlockSpec((_BE, _HID), lambda i: (i, 0)),
            pl.BlockSpec((_BE, _HID), lambda i: (i, 0)),
            pl.BlockSpec((_BE, zdim - 2 * _HID), lambda i: (i, 0)),
            pl.BlockSpec((zdim, _HID), lambda i: (0, 0)),
            pl.BlockSpec((zdim, _HID), lambda i: (0, 0)),
            pl.BlockSpec((1, _HID), lambda i: (0, 0)),
            pl.BlockSpec((1, _HID), lambda i: (0, 0)),
        ],
        out_specs=pl.BlockSpec((_BE, _HID), lambda i: (i, 0)),
        out_shape=jax.ShapeDtypeStruct((e, _HID), jnp.float32),
    )(hs, hd, attr, ws_t, wm_t, bsi.reshape(1, _HID), bmi.reshape(1, _HID))


def kernel(x, edge_index, edge_attr, W_emb, b_emb, Ws, bs, Wm, bm, gamma,
           beta, W_fc, b_fc, W_out, b_out):
    src = edge_index[0]
    dst = edge_index[1]
    h = x @ W_emb.T + b_emb
    for i in range(_NCONV):
        hs, hd = _gather2(h, src, dst)
        msg = _msg(hs, hd, edge_attr, Ws[i].T, Wm[i].T, bs[i], bm[i])
        h = h + jax.ops.segment_sum(msg, dst, num_segments=_N)
        mean = jnp.mean(h, axis=0)
        var = jnp.var(h, axis=0)
        h = (h - mean) / jnp.sqrt(var + 1e-5) * gamma[i] + beta[i]
    graph_feat = jnp.sum(h, axis=0, keepdims=True)
    graph_feat = graph_feat @ W_fc.T + b_fc
    out = graph_feat @ W_out.T + b_out
    return out
